# baseline (speedup 1.0000x reference)
import functools

import jax
import jax.numpy as jnp
from jax import lax
from jax.experimental import pallas as pl
from jax.experimental.pallas import tpu as pltpu
from jax.experimental.pallas import tpu_sc as plsc


def _make_gather(B, V, D):
    info = plsc.get_sparse_core_info()
    nw = info.num_cores * info.num_subcores
    b_per_w = B // nw
    mesh = plsc.VectorSubcoreMesh(core_axis_name="c", subcore_axis_name="s")

    @functools.partial(
        pl.kernel,
        mesh=mesh,
        out_type=jax.ShapeDtypeStruct((B, 2 * D), jnp.float32),
        scratch_types=[
            pltpu.VMEM((b_per_w,), jnp.int32),
            pltpu.VMEM((b_per_w, 2 * D), jnp.float32),
            pltpu.SemaphoreType.DMA,
        ],
    )
    def gather_kernel(idx_hbm, tbl_hbm, out_hbm, idx_v, rows_v, sem):
        wid = lax.axis_index("s") * info.num_cores + lax.axis_index("c")
        base = wid * b_per_w
        pltpu.sync_copy(idx_hbm.at[pl.ds(base, b_per_w)], idx_v)
        pltpu.async_copy(tbl_hbm.at[idx_v], rows_v, sem).wait()
        pltpu.sync_copy(rows_v, out_hbm.at[pl.ds(base, b_per_w)])

    return gather_kernel


def kernel(target_spk, life_long_mem):
    idx = jnp.reshape(target_spk, (target_spk.shape[0],)).astype(jnp.int32)
    B = idx.shape[0]
    V, D = life_long_mem.shape
    tbl_wide = jnp.concatenate([life_long_mem, life_long_mem], axis=1)
    return _make_gather(B, V, D)(idx, tbl_wide)[:, :D]


# speedup vs baseline: 2.4426x; 2.4426x over previous
import functools

import jax
import jax.numpy as jnp
from jax import lax
from jax.experimental import pallas as pl
from jax.experimental.pallas import tpu as pltpu
from jax.experimental.pallas import tpu_sc as plsc

_BL = 16384


def _transpose_pack(tbl_t):
    D, V = tbl_t.shape
    H = _BL // 2
    nb = pl.cdiv(V, _BL)

    def body(in_ref, out_ref):
        eye = jnp.eye(D, dtype=jnp.float32)
        out_ref[:, 0:D] = jnp.swapaxes(in_ref[:, 0:H], 0, 1)
        out_ref[:, D : 2 * D] = lax.dot_general(
            in_ref[:, H:_BL],
            eye,
            dimension_numbers=(((0,), (0,)), ((), ())),
            precision=lax.Precision.HIGHEST,
            preferred_element_type=jnp.float32,
        )

    return pl.pallas_call(
        body,
        grid=(nb,),
        in_specs=[pl.BlockSpec((D, _BL), lambda c: (0, c))],
        out_specs=pl.BlockSpec((H, 2 * D), lambda c: (c, 0)),
        out_shape=jax.ShapeDtypeStruct((nb * H, 2 * D), jnp.float32),
    )(tbl_t)


def _make_gather(B, D):
    info = plsc.get_sparse_core_info()
    nw = info.num_cores * info.num_subcores
    b_per_w = B // nw
    mesh = plsc.VectorSubcoreMesh(core_axis_name="c", subcore_axis_name="s")

    @functools.partial(
        pl.kernel,
        mesh=mesh,
        out_type=jax.ShapeDtypeStruct((B, 2 * D), jnp.float32),
        scratch_types=[
            pltpu.VMEM((b_per_w,), jnp.int32),
            pltpu.VMEM((b_per_w, 2 * D), jnp.float32),
            pltpu.SemaphoreType.DMA,
        ],
    )
    def gather_kernel(pair_hbm, tbl_hbm, out_hbm, pair_v, rows_v, sem):
        wid = lax.axis_index("s") * info.num_cores + lax.axis_index("c")
        base = wid * b_per_w
        pltpu.sync_copy(pair_hbm.at[pl.ds(base, b_per_w)], pair_v)
        pltpu.async_copy(tbl_hbm.at[pair_v], rows_v, sem).wait()
        pltpu.sync_copy(rows_v, out_hbm.at[pl.ds(base, b_per_w)])

    return gather_kernel


def _select_half(pairs, half):
    B, W = pairs.shape
    D = W // 2
    blk = 512

    def body(half_ref, rows_ref, out_ref):
        rows = rows_ref[...]
        out_ref[...] = jnp.where(half_ref[...] == 1, rows[:, D:W], rows[:, 0:D])

    return pl.pallas_call(
        body,
        grid=(B // blk,),
        in_specs=[
            pl.BlockSpec((blk, 1), lambda i: (i, 0)),
            pl.BlockSpec((blk, W), lambda i: (i, 0)),
        ],
        out_specs=pl.BlockSpec((blk, D), lambda i: (i, 0)),
        out_shape=jax.ShapeDtypeStruct((B, D), jnp.float32),
    )(half, pairs)


def kernel(target_spk, life_long_mem):
    idx = jnp.reshape(target_spk, (target_spk.shape[0],)).astype(jnp.int32)
    B = idx.shape[0]
    V, D = life_long_mem.shape
    H = _BL // 2
    o = idx % _BL
    pair = (idx // _BL) * H + (o % H)
    half = jnp.reshape((o // H).astype(jnp.int32), (B, 1))
    packed = _transpose_pack(life_long_mem.T)
    pairs = _make_gather(B, D)(pair, packed)
    return _select_half(pairs, half)


# speedup vs baseline: 2.5137x; 1.0291x over previous
import functools

import jax
import jax.numpy as jnp
from jax import lax
from jax.experimental import pallas as pl
from jax.experimental.pallas import tpu as pltpu
from jax.experimental.pallas import tpu_sc as plsc

_BL = 16384


def _transpose_pack(tbl_t):
    D, V = tbl_t.shape
    H = _BL // 2
    nb = pl.cdiv(V, _BL)

    S = 11264

    def body(in_ref, out_ref):
        eye = jnp.eye(D, dtype=jnp.float32)
        ta = jnp.swapaxes(in_ref[:, 0:S], 0, 1)
        tb = lax.dot_general(
            in_ref[:, S:_BL],
            eye,
            dimension_numbers=(((0,), (0,)), ((), ())),
            precision=lax.Precision.HIGHEST,
            preferred_element_type=jnp.float32,
        )
        out_ref[:, 0:D] = ta[0:H]
        out_ref[0 : S - H, D : 2 * D] = ta[H:S]
        out_ref[S - H : H, D : 2 * D] = tb

    return pl.pallas_call(
        body,
        grid=(nb,),
        in_specs=[pl.BlockSpec((D, _BL), lambda c: (0, c))],
        out_specs=pl.BlockSpec((H, 2 * D), lambda c: (c, 0)),
        out_shape=jax.ShapeDtypeStruct((nb * H, 2 * D), jnp.float32),
    )(tbl_t)


def _make_gather(B, D):
    info = plsc.get_sparse_core_info()
    nw = info.num_cores * info.num_subcores
    b_per_w = B // nw
    mesh = plsc.VectorSubcoreMesh(core_axis_name="c", subcore_axis_name="s")

    @functools.partial(
        pl.kernel,
        mesh=mesh,
        out_type=jax.ShapeDtypeStruct((B, 2 * D), jnp.float32),
        scratch_types=[
            pltpu.VMEM((b_per_w,), jnp.int32),
            pltpu.VMEM((b_per_w, 2 * D), jnp.float32),
            pltpu.SemaphoreType.DMA,
        ],
    )
    def gather_kernel(pair_hbm, tbl_hbm, out_hbm, pair_v, rows_v, sem):
        wid = lax.axis_index("s") * info.num_cores + lax.axis_index("c")
        base = wid * b_per_w
        pltpu.sync_copy(pair_hbm.at[pl.ds(base, b_per_w)], pair_v)
        pltpu.async_copy(tbl_hbm.at[pair_v], rows_v, sem).wait()
        pltpu.sync_copy(rows_v, out_hbm.at[pl.ds(base, b_per_w)])

    return gather_kernel


def _select_half(pairs, half):
    B, W = pairs.shape
    D = W // 2
    blk = 512

    def body(half_ref, rows_ref, out_ref):
        rows = rows_ref[...]
        out_ref[...] = jnp.where(half_ref[...] == 1, rows[:, D:W], rows[:, 0:D])

    return pl.pallas_call(
        body,
        grid=(B // blk,),
        in_specs=[
            pl.BlockSpec((blk, 1), lambda i: (i, 0)),
            pl.BlockSpec((blk, W), lambda i: (i, 0)),
        ],
        out_specs=pl.BlockSpec((blk, D), lambda i: (i, 0)),
        out_shape=jax.ShapeDtypeStruct((B, D), jnp.float32),
    )(half, pairs)


def kernel(target_spk, life_long_mem):
    idx = jnp.reshape(target_spk, (target_spk.shape[0],)).astype(jnp.int32)
    B = idx.shape[0]
    V, D = life_long_mem.shape
    H = _BL // 2
    o = idx % _BL
    pair = (idx // _BL) * H + (o % H)
    half = jnp.reshape((o // H).astype(jnp.int32), (B, 1))
    packed = _transpose_pack(life_long_mem.T)
    pairs = _make_gather(B, D)(pair, packed)
    return _select_half(pairs, half)


# speedup vs baseline: 2.6713x; 1.0627x over previous
import functools

import jax
import jax.numpy as jnp
from jax import lax
from jax.experimental import pallas as pl
from jax.experimental.pallas import tpu as pltpu
from jax.experimental.pallas import tpu_sc as plsc

_BL = 32768


def _transpose_pack(tbl_t):
    D, V = tbl_t.shape
    H = _BL // 2
    nb = pl.cdiv(V, _BL)

    def body(in_ref, out_ref):
        t = jnp.swapaxes(in_ref[...], 0, 1)
        out_ref[:, 0:D] = t[0:H]
        out_ref[:, D : 2 * D] = t[H:_BL]

    return pl.pallas_call(
        body,
        grid=(nb,),
        in_specs=[pl.BlockSpec((D, _BL), lambda c: (0, c))],
        out_specs=pl.BlockSpec((H, 2 * D), lambda c: (c, 0)),
        out_shape=jax.ShapeDtypeStruct((nb * H, 2 * D), jnp.float32),
    )(tbl_t)


def _make_gather(B, D):
    info = plsc.get_sparse_core_info()
    nw = info.num_cores * info.num_subcores
    b_per_w = B // nw
    mesh = plsc.VectorSubcoreMesh(core_axis_name="c", subcore_axis_name="s")

    @functools.partial(
        pl.kernel,
        mesh=mesh,
        out_type=jax.ShapeDtypeStruct((B, 2 * D), jnp.float32),
        scratch_types=[
            pltpu.VMEM((b_per_w,), jnp.int32),
            pltpu.VMEM((b_per_w, 2 * D), jnp.float32),
            pltpu.SemaphoreType.DMA,
        ],
    )
    def gather_kernel(pair_hbm, tbl_hbm, out_hbm, pair_v, rows_v, sem):
        wid = lax.axis_index("s") * info.num_cores + lax.axis_index("c")
        base = wid * b_per_w
        pltpu.sync_copy(pair_hbm.at[pl.ds(base, b_per_w)], pair_v)
        pltpu.async_copy(tbl_hbm.at[pair_v], rows_v, sem).wait()
        pltpu.sync_copy(rows_v, out_hbm.at[pl.ds(base, b_per_w)])

    return gather_kernel


def _select_half(pairs, half):
    B, W = pairs.shape
    D = W // 2
    blk = 512

    def body(half_ref, rows_ref, out_ref):
        rows = rows_ref[...]
        out_ref[...] = jnp.where(half_ref[...] == 1, rows[:, D:W], rows[:, 0:D])

    return pl.pallas_call(
        body,
        grid=(B // blk,),
        in_specs=[
            pl.BlockSpec((blk, 1), lambda i: (i, 0)),
            pl.BlockSpec((blk, W), lambda i: (i, 0)),
        ],
        out_specs=pl.BlockSpec((blk, D), lambda i: (i, 0)),
        out_shape=jax.ShapeDtypeStruct((B, D), jnp.float32),
    )(half, pairs)


def kernel(target_spk, life_long_mem):
    idx = jnp.reshape(target_spk, (target_spk.shape[0],)).astype(jnp.int32)
    B = idx.shape[0]
    V, D = life_long_mem.shape
    H = _BL // 2
    o = idx % _BL
    pair = (idx // _BL) * H + (o % H)
    half = jnp.reshape((o // H).astype(jnp.int32), (B, 1))
    packed = _transpose_pack(life_long_mem.T)
    pairs = _make_gather(B, D)(pair, packed)
    return _select_half(pairs, half)
